# Initial kernel scaffold; baseline (speedup 1.0000x reference)
import functools

import jax
import jax.numpy as jnp
from jax import lax
from jax.experimental import pallas as pl
from jax.experimental.pallas import tpu as pltpu
from jax.experimental.pallas import tpu_sc as plsc

NC = 2
NS = 16
NW = NC * NS
WIN = 128

_MESH = plsc.VectorSubcoreMesh(
    core_axis_name="c", subcore_axis_name="s", num_cores=NC, num_subcores=NS
)


def _deg_kernel_make(n_rows, wpw):
    stripe = n_rows // NS

    @functools.partial(
        pl.kernel,
        out_type=jax.ShapeDtypeStruct((NC * n_rows, 16), jnp.float32),
        mesh=_MESH,
        scratch_types=[
            pltpu.VMEM((wpw, WIN), jnp.int32),
            pltpu.VMEM((WIN, 16), jnp.float32),
            pltpu.VMEM((WIN, 16), jnp.float32),
            pltpu.VMEM_SHARED((n_rows, 16), jnp.float32),
        ],
    )
    def deg_kernel(dst_hbm, out_hbm, dstbuf, ones, zeros, hist):
        c = lax.axis_index("c")
        s = lax.axis_index("s")

        @pl.loop(0, WIN)
        def _(j):
            ones[j, :] = jnp.full((16,), 1.0, jnp.float32)
            zeros[j, :] = jnp.zeros((16,), jnp.float32)

        @pl.loop(0, stripe // WIN)
        def _(k):
            pltpu.sync_copy(zeros, hist.at[pl.ds(s * stripe + k * WIN, WIN)])

        base = (c * NS + s) * wpw
        pltpu.sync_copy(dst_hbm.at[pl.ds(base, wpw)], dstbuf)
        plsc.subcore_barrier()

        @pl.loop(0, wpw)
        def _(j):
            pltpu.sync_copy(ones, hist.at[dstbuf.at[j]], add=True)

        plsc.subcore_barrier()
        pltpu.sync_copy(
            hist.at[pl.ds(s * stripe, stripe)],
            out_hbm.at[pl.ds(c * n_rows + s * stripe, stripe)],
        )

    return deg_kernel


def _edge_kernel_make(n, n_rows, d, wpw):
    stripe = n // NS

    @functools.partial(
        pl.kernel,
        out_type=jax.ShapeDtypeStruct((NC * n, d), jnp.float32),
        mesh=_MESH,
        scratch_types=[
            pltpu.VMEM((wpw, WIN), jnp.int32),
            pltpu.VMEM((wpw, WIN), jnp.int32),
            pltpu.VMEM((WIN, d), jnp.float32),
            pltpu.VMEM((WIN, d), jnp.float32),
            pltpu.VMEM_SHARED((n_rows, d), jnp.float32),
            pltpu.SemaphoreType.DMA,
            pltpu.SemaphoreType.DMA,
        ],
    )
    def edge_kernel(hp_hbm, src_hbm, dst_hbm, out_hbm, srcbuf, dstbuf,
                    rbuf0, rbuf1, acc, sem0, sem1):
        c = lax.axis_index("c")
        s = lax.axis_index("s")
        base = (c * NS + s) * wpw

        pltpu.sync_copy(src_hbm.at[pl.ds(base, wpw)], srcbuf)
        pltpu.sync_copy(dst_hbm.at[pl.ds(base, wpw)], dstbuf)
        pltpu.sync_copy(
            hp_hbm.at[pl.ds(s * stripe, stripe)],
            acc.at[pl.ds(s * stripe, stripe)],
        )
        pltpu.async_copy(hp_hbm.at[srcbuf.at[0]], rbuf0, sem0)
        pltpu.async_copy(hp_hbm.at[srcbuf.at[1]], rbuf1, sem1)
        plsc.subcore_barrier()

        @pl.loop(0, wpw, step=2)
        def _(j):
            pltpu.make_async_copy(hp_hbm.at[srcbuf.at[j]], rbuf0, sem0).wait()
            pltpu.sync_copy(rbuf0, acc.at[dstbuf.at[j]], add=True)

            @pl.when(j + 2 < wpw)
            def _():
                pltpu.async_copy(hp_hbm.at[srcbuf.at[j + 2]], rbuf0, sem0)

            pltpu.make_async_copy(hp_hbm.at[srcbuf.at[j + 1]], rbuf1, sem1).wait()
            pltpu.sync_copy(rbuf1, acc.at[dstbuf.at[j + 1]], add=True)

            @pl.when(j + 3 < wpw)
            def _():
                pltpu.async_copy(hp_hbm.at[srcbuf.at[j + 3]], rbuf1, sem1)

        plsc.subcore_barrier()
        pltpu.sync_copy(
            acc.at[pl.ds(s * stripe, stripe)],
            out_hbm.at[pl.ds(c * n + s * stripe, stripe)],
        )

    return edge_kernel


def _matmul(x, w):
    m, k = x.shape
    d = w.shape[1]
    blk = 2000

    def body(x_ref, w_ref, o_ref):
        o_ref[...] = jnp.dot(x_ref[...], w_ref[...],
                             preferred_element_type=jnp.float32)

    return pl.pallas_call(
        body,
        grid=(m // blk,),
        in_specs=[
            pl.BlockSpec((blk, k), lambda i: (i, 0)),
            pl.BlockSpec((k, d), lambda i: (0, 0)),
        ],
        out_specs=pl.BlockSpec((blk, d), lambda i: (i, 0)),
        out_shape=jax.ShapeDtypeStruct((m, d), jnp.float32),
    )(x, w)


def _scale(d0, d1, h):
    n, d = h.shape
    blk = 2000

    def body(d0_ref, d1_ref, h_ref, hp_ref, dis_ref):
        dis = lax.rsqrt(d0_ref[...] + d1_ref[...] + 1.0)
        dis_ref[...] = dis
        hp_ref[...] = h_ref[...] * dis

    return pl.pallas_call(
        body,
        grid=(n // blk,),
        in_specs=[
            pl.BlockSpec((blk, 1), lambda i: (i, 0)),
            pl.BlockSpec((blk, 1), lambda i: (i, 0)),
            pl.BlockSpec((blk, d), lambda i: (i, 0)),
        ],
        out_specs=[
            pl.BlockSpec((blk, d), lambda i: (i, 0)),
            pl.BlockSpec((blk, 1), lambda i: (i, 0)),
        ],
        out_shape=[
            jax.ShapeDtypeStruct((n, d), jnp.float32),
            jax.ShapeDtypeStruct((n, 1), jnp.float32),
        ],
    )(d0, d1, h)


def _finalize(p0, p1, hp, dis, b):
    n, d = hp.shape
    blk = 2000

    def body(p0_ref, p1_ref, hp_ref, dis_ref, b_ref, o_ref):
        agg = p0_ref[...] + p1_ref[...] - hp_ref[...]
        o_ref[...] = jnp.maximum(dis_ref[...] * agg + b_ref[...], 0.0)

    return pl.pallas_call(
        body,
        grid=(n // blk,),
        in_specs=[
            pl.BlockSpec((blk, d), lambda i: (i, 0)),
            pl.BlockSpec((blk, d), lambda i: (i, 0)),
            pl.BlockSpec((blk, d), lambda i: (i, 0)),
            pl.BlockSpec((blk, 1), lambda i: (i, 0)),
            pl.BlockSpec((1, d), lambda i: (0, 0)),
        ],
        out_specs=pl.BlockSpec((blk, d), lambda i: (i, 0)),
        out_shape=jax.ShapeDtypeStruct((n, d), jnp.float32),
    )(p0, p1, hp, dis, b)


def kernel(x, edge_index, W, b):
    n, _ = x.shape
    d = W.shape[1]
    e = edge_index.shape[1]

    ep = -(-e // (2 * NW * WIN)) * (2 * NW * WIN)
    wpw = ep // (NW * WIN)
    n_rows = -(-(n + 1) // (NS * WIN)) * (NS * WIN)

    src = edge_index[0]
    dst = edge_index[1]
    src_p = jnp.concatenate(
        [src, jnp.zeros((ep - e,), jnp.int32)]).reshape(NW * wpw, WIN)
    dst_p = jnp.concatenate(
        [dst, jnp.full((ep - e,), n, jnp.int32)]).reshape(NW * wpw, WIN)

    hist = _deg_kernel_make(n_rows, wpw)(dst_p)
    h = _matmul(x, W)

    d0 = hist[:n, 0:1]
    d1 = hist[n_rows:n_rows + n, 0:1]
    hp, dis = _scale(d0, d1, h)

    parts = _edge_kernel_make(n, n_rows, d, wpw)(hp, src_p, dst_p)
    out = _finalize(parts[:n], parts[n:], hp, dis, b.reshape(1, d))
    return out


# speedup vs baseline: 12.0310x; 12.0310x over previous
import functools

import jax
import jax.numpy as jnp
from jax import lax
from jax.experimental import pallas as pl
from jax.experimental.pallas import tpu as pltpu
from jax.experimental.pallas import tpu_sc as plsc

NC = 2
NS = 16
NW = NC * NS
WIN = 128

_MESH = plsc.VectorSubcoreMesh(
    core_axis_name="c", subcore_axis_name="s", num_cores=NC, num_subcores=NS
)


def _deg_kernel_make(np_, wpw):
    stripe = np_ // NS

    @functools.partial(
        pl.kernel,
        out_type=jax.ShapeDtypeStruct((NC * np_, 16), jnp.float32),
        mesh=_MESH,
        scratch_types=[
            pltpu.VMEM((wpw, WIN), jnp.int32),
            pltpu.VMEM((WIN, 16), jnp.float32),
            pltpu.VMEM((WIN, 16), jnp.float32),
            pltpu.VMEM_SHARED((np_, 16), jnp.float32),
        ],
    )
    def deg_kernel(dst_hbm, out_hbm, dstbuf, ones, zeros, hist):
        c = lax.axis_index("c")
        s = lax.axis_index("s")

        @pl.loop(0, WIN)
        def _(j):
            ones[j, :] = jnp.full((16,), 1.0, jnp.float32)
            zeros[j, :] = jnp.zeros((16,), jnp.float32)

        @pl.loop(0, stripe // WIN)
        def _(k):
            pltpu.sync_copy(zeros, hist.at[pl.ds(s * stripe + k * WIN, WIN)])

        base = (c * NS + s) * wpw
        pltpu.sync_copy(dst_hbm.at[pl.ds(base, wpw)], dstbuf)
        plsc.subcore_barrier()

        @pl.loop(0, wpw)
        def _(j):
            pltpu.sync_copy(ones, hist.at[dstbuf.at[j]], add=True)

        plsc.subcore_barrier()
        pltpu.sync_copy(
            hist.at[pl.ds(s * stripe, stripe)],
            out_hbm.at[pl.ds(c * np_ + s * stripe, stripe)],
        )

    return deg_kernel


def _edge_kernel_make(np_, d, wpw):
    stripe = np_ // NS
    half = wpw // 2
    assert half % 2 == 0

    @functools.partial(
        pl.kernel,
        out_type=jax.ShapeDtypeStruct((NC * np_, d), jnp.float32),
        mesh=_MESH,
        scratch_types=[
            pltpu.VMEM((half, WIN), jnp.int32),
            pltpu.VMEM((half, WIN), jnp.int32),
            pltpu.VMEM((WIN, d), jnp.float32),
            pltpu.VMEM((WIN, d), jnp.float32),
            pltpu.VMEM_SHARED((np_, d), jnp.float32),
            pltpu.SemaphoreType.DMA,
            pltpu.SemaphoreType.DMA,
        ],
    )
    def edge_kernel(hp_hbm, src_hbm, dst_hbm, out_hbm, srcbuf, dstbuf,
                    rbuf0, rbuf1, acc, sem0, sem1):
        c = lax.axis_index("c")
        s = lax.axis_index("s")
        base = (c * NS + s) * wpw

        pltpu.sync_copy(
            hp_hbm.at[pl.ds(s * stripe, stripe)],
            acc.at[pl.ds(s * stripe, stripe)],
        )
        for hix in range(2):
            pltpu.sync_copy(src_hbm.at[pl.ds(base + hix * half, half)], srcbuf)
            pltpu.sync_copy(dst_hbm.at[pl.ds(base + hix * half, half)], dstbuf)
            pltpu.async_copy(hp_hbm.at[srcbuf.at[0]], rbuf0, sem0)
            pltpu.async_copy(hp_hbm.at[srcbuf.at[1]], rbuf1, sem1)
            if hix == 0:
                plsc.subcore_barrier()

            @pl.loop(0, half, step=2)
            def _(j):
                pltpu.make_async_copy(
                    hp_hbm.at[srcbuf.at[j]], rbuf0, sem0).wait()
                pltpu.sync_copy(rbuf0, acc.at[dstbuf.at[j]], add=True)

                @pl.when(j + 2 < half)
                def _():
                    pltpu.async_copy(hp_hbm.at[srcbuf.at[j + 2]], rbuf0, sem0)

                pltpu.make_async_copy(
                    hp_hbm.at[srcbuf.at[j + 1]], rbuf1, sem1).wait()
                pltpu.sync_copy(rbuf1, acc.at[dstbuf.at[j + 1]], add=True)

                @pl.when(j + 3 < half)
                def _():
                    pltpu.async_copy(hp_hbm.at[srcbuf.at[j + 3]], rbuf1, sem1)

        plsc.subcore_barrier()
        pltpu.sync_copy(
            acc.at[pl.ds(s * stripe, stripe)],
            out_hbm.at[pl.ds(c * np_ + s * stripe, stripe)],
        )

    return edge_kernel


def _matmul(x, w):
    m, k = x.shape
    d = w.shape[1]
    blk = 2048

    def body(x_ref, w_ref, o_ref):
        o_ref[...] = jnp.dot(x_ref[...], w_ref[...],
                             preferred_element_type=jnp.float32)

    return pl.pallas_call(
        body,
        grid=(m // blk,),
        in_specs=[
            pl.BlockSpec((blk, k), lambda i: (i, 0)),
            pl.BlockSpec((k, d), lambda i: (0, 0)),
        ],
        out_specs=pl.BlockSpec((blk, d), lambda i: (i, 0)),
        out_shape=jax.ShapeDtypeStruct((m, d), jnp.float32),
    )(x, w)


def _scale(hist, h):
    np_, d = h.shape
    blk = 2048
    off1 = np_ // blk

    def body(h0_ref, h1_ref, h_ref, hp_ref, dis_ref):
        deg = h0_ref[..., 0:1] + h1_ref[..., 0:1] + 1.0
        dis = lax.rsqrt(deg)
        dis_ref[...] = dis
        hp_ref[...] = h_ref[...] * dis

    return pl.pallas_call(
        body,
        grid=(np_ // blk,),
        in_specs=[
            pl.BlockSpec((blk, 16), lambda i: (i, 0)),
            pl.BlockSpec((blk, 16), lambda i: (i + off1, 0)),
            pl.BlockSpec((blk, d), lambda i: (i, 0)),
        ],
        out_specs=[
            pl.BlockSpec((blk, d), lambda i: (i, 0)),
            pl.BlockSpec((blk, 1), lambda i: (i, 0)),
        ],
        out_shape=[
            jax.ShapeDtypeStruct((np_, d), jnp.float32),
            jax.ShapeDtypeStruct((np_, 1), jnp.float32),
        ],
    )(hist, hist, h)


def _finalize(n, np_, parts, hp, dis, b):
    d = hp.shape[1]
    blk = 80
    off1 = np_ // blk

    def body(p0_ref, p1_ref, hp_ref, dis_ref, b_ref, o_ref):
        agg = p0_ref[...] + p1_ref[...] - hp_ref[...]
        o_ref[...] = jnp.maximum(dis_ref[...] * agg + b_ref[...], 0.0)

    return pl.pallas_call(
        body,
        grid=(n // blk,),
        in_specs=[
            pl.BlockSpec((blk, d), lambda i: (i, 0)),
            pl.BlockSpec((blk, d), lambda i: (i + off1, 0)),
            pl.BlockSpec((blk, d), lambda i: (i, 0)),
            pl.BlockSpec((blk, 1), lambda i: (i, 0)),
            pl.BlockSpec((1, d), lambda i: (0, 0)),
        ],
        out_specs=pl.BlockSpec((blk, d), lambda i: (i, 0)),
        out_shape=jax.ShapeDtypeStruct((n, d), jnp.float32),
    )(parts, parts, hp, dis, b)


def kernel(x, edge_index, W, b):
    n, _ = x.shape
    d = W.shape[1]
    e = edge_index.shape[1]

    ep = -(-e // (2 * NW * WIN)) * (2 * NW * WIN)
    wpw = ep // (NW * WIN)
    np_ = -(-(n + 1) // (NS * WIN)) * (NS * WIN)

    src = edge_index[0]
    dst = edge_index[1]
    src_p = jnp.concatenate(
        [src, jnp.zeros((ep - e,), jnp.int32)]).reshape(NW * wpw, WIN)
    dst_p = jnp.concatenate(
        [dst, jnp.full((ep - e,), n, jnp.int32)]).reshape(NW * wpw, WIN)
    x_p = jnp.pad(x, ((0, np_ - n), (0, 0)))

    hist = _deg_kernel_make(np_, wpw)(dst_p)
    h = _matmul(x_p, W)
    hp, dis = _scale(hist, h)
    parts = _edge_kernel_make(np_, d, wpw)(hp, src_p, dst_p)
    return _finalize(n, np_, parts, hp, dis, b.reshape(1, d))


# speedup vs baseline: 12.4335x; 1.0335x over previous
import functools

import jax
import jax.numpy as jnp
from jax import lax
from jax.experimental import pallas as pl
from jax.experimental.pallas import tpu as pltpu
from jax.experimental.pallas import tpu_sc as plsc

NC = 2
NS = 16
NW = NC * NS
WIN = 128
CHUNK = 40

_MESH = plsc.VectorSubcoreMesh(
    core_axis_name="c", subcore_axis_name="s", num_cores=NC, num_subcores=NS
)


def _deg_kernel_make(np_, wpw):
    stripe = np_ // NS

    @functools.partial(
        pl.kernel,
        out_type=[jax.ShapeDtypeStruct((np_, 16), jnp.float32)
                  for _ in range(NC)],
        mesh=_MESH,
        scratch_types=[
            pltpu.VMEM((wpw, WIN), jnp.int32),
            pltpu.VMEM((WIN, 16), jnp.float32),
            pltpu.VMEM((WIN, 16), jnp.float32),
            pltpu.VMEM_SHARED((np_, 16), jnp.float32),
        ],
    )
    def deg_kernel(dst_hbm, *out_and_scratch):
        outs = out_and_scratch[:NC]
        dstbuf, ones, zeros, hist = out_and_scratch[NC:]
        c = lax.axis_index("c")
        s = lax.axis_index("s")

        @pl.loop(0, WIN)
        def _(j):
            ones[j, :] = jnp.full((16,), 1.0, jnp.float32)
            zeros[j, :] = jnp.zeros((16,), jnp.float32)

        @pl.loop(0, stripe // WIN)
        def _(k):
            pltpu.sync_copy(zeros, hist.at[pl.ds(s * stripe + k * WIN, WIN)])

        base = (c * NS + s) * wpw
        pltpu.sync_copy(dst_hbm.at[pl.ds(base, wpw)], dstbuf)
        plsc.subcore_barrier()

        @pl.loop(0, wpw)
        def _(j):
            pltpu.sync_copy(ones, hist.at[dstbuf.at[j]], add=True)

        plsc.subcore_barrier()
        for ci in range(NC):
            @pl.when(c == ci)
            def _():
                pltpu.sync_copy(hist.at[pl.ds(s * stripe, stripe)],
                                outs[ci].at[pl.ds(s * stripe, stripe)])

    return deg_kernel


def _edge_kernel_make(np_, d, wpw):
    stripe = np_ // NS
    assert wpw % CHUNK == 0 and CHUNK % 2 == 0

    @functools.partial(
        pl.kernel,
        out_type=[jax.ShapeDtypeStruct((np_, d), jnp.float32)
                  for _ in range(NC)],
        mesh=_MESH,
        scratch_types=[
            pltpu.VMEM((CHUNK, WIN), jnp.int32),
            pltpu.VMEM((CHUNK, WIN), jnp.int32),
            pltpu.VMEM((WIN, d), jnp.float32),
            pltpu.VMEM((WIN, d), jnp.float32),
            pltpu.VMEM_SHARED((np_, d), jnp.float32),
            pltpu.SemaphoreType.DMA,
            pltpu.SemaphoreType.DMA,
        ],
    )
    def edge_kernel(hp_hbm, src_hbm, dst_hbm, *out_and_scratch):
        outs = out_and_scratch[:NC]
        srcbuf, dstbuf, rbuf0, rbuf1, acc, sem0, sem1 = out_and_scratch[NC:]
        c = lax.axis_index("c")
        s = lax.axis_index("s")
        base = (c * NS + s) * wpw

        pltpu.sync_copy(
            hp_hbm.at[pl.ds(s * stripe, stripe)],
            acc.at[pl.ds(s * stripe, stripe)],
        )
        for hix in range(wpw // CHUNK):
            pltpu.sync_copy(
                src_hbm.at[pl.ds(base + hix * CHUNK, CHUNK)], srcbuf)
            pltpu.sync_copy(
                dst_hbm.at[pl.ds(base + hix * CHUNK, CHUNK)], dstbuf)
            pltpu.async_copy(hp_hbm.at[srcbuf.at[0]], rbuf0, sem0)
            pltpu.async_copy(hp_hbm.at[srcbuf.at[1]], rbuf1, sem1)
            if hix == 0:
                plsc.subcore_barrier()

            @pl.loop(0, CHUNK, step=2)
            def _(j):
                pltpu.make_async_copy(
                    hp_hbm.at[srcbuf.at[j]], rbuf0, sem0).wait()
                pltpu.sync_copy(rbuf0, acc.at[dstbuf.at[j]], add=True)

                @pl.when(j + 2 < CHUNK)
                def _():
                    pltpu.async_copy(hp_hbm.at[srcbuf.at[j + 2]], rbuf0, sem0)

                pltpu.make_async_copy(
                    hp_hbm.at[srcbuf.at[j + 1]], rbuf1, sem1).wait()
                pltpu.sync_copy(rbuf1, acc.at[dstbuf.at[j + 1]], add=True)

                @pl.when(j + 3 < CHUNK)
                def _():
                    pltpu.async_copy(hp_hbm.at[srcbuf.at[j + 3]], rbuf1, sem1)

        plsc.subcore_barrier()
        for ci in range(NC):
            @pl.when(c == ci)
            def _():
                pltpu.sync_copy(acc.at[pl.ds(s * stripe, stripe)],
                                outs[ci].at[pl.ds(s * stripe, stripe)])

    return edge_kernel


def _matmul(x, w):
    m, k = x.shape
    d = w.shape[1]
    blk = 2048

    def body(x_ref, w_ref, o_ref):
        o_ref[...] = jnp.dot(x_ref[...], w_ref[...],
                             preferred_element_type=jnp.float32)

    return pl.pallas_call(
        body,
        grid=(m // blk,),
        in_specs=[
            pl.BlockSpec((blk, k), lambda i: (i, 0)),
            pl.BlockSpec((k, d), lambda i: (0, 0)),
        ],
        out_specs=pl.BlockSpec((blk, d), lambda i: (i, 0)),
        out_shape=jax.ShapeDtypeStruct((m, d), jnp.float32),
    )(x, w)


def _scale(hists, h):
    np_, d = h.shape
    blk = 2048

    def body(*refs):
        hist_refs = refs[:NC]
        h_ref, hp_ref, dis_ref = refs[NC:]
        deg = hist_refs[0][..., 0:1] + 1.0
        for r in hist_refs[1:]:
            deg = deg + r[..., 0:1]
        dis = lax.rsqrt(deg)
        dis_ref[...] = dis
        hp_ref[...] = h_ref[...] * dis

    return pl.pallas_call(
        body,
        grid=(np_ // blk,),
        in_specs=[pl.BlockSpec((blk, 16), lambda i: (i, 0))
                  for _ in range(NC)] + [
            pl.BlockSpec((blk, d), lambda i: (i, 0)),
        ],
        out_specs=[
            pl.BlockSpec((blk, d), lambda i: (i, 0)),
            pl.BlockSpec((blk, 1), lambda i: (i, 0)),
        ],
        out_shape=[
            jax.ShapeDtypeStruct((np_, d), jnp.float32),
            jax.ShapeDtypeStruct((np_, 1), jnp.float32),
        ],
    )(*hists, h)


def _finalize(n, parts, hp, dis, b):
    d = hp.shape[1]
    blk = 2000

    def body(*refs):
        p_refs = refs[:NC]
        hp_ref, dis_ref, b_ref, o_ref = refs[NC:]
        agg = p_refs[0][...]
        for r in p_refs[1:]:
            agg = agg + r[...]
        agg = agg - (NC - 1) * hp_ref[...]
        o_ref[...] = jnp.maximum(dis_ref[...] * agg + b_ref[...], 0.0)

    return pl.pallas_call(
        body,
        grid=(n // blk,),
        in_specs=[pl.BlockSpec((blk, d), lambda i: (i, 0))
                  for _ in range(NC)] + [
            pl.BlockSpec((blk, d), lambda i: (i, 0)),
            pl.BlockSpec((blk, 1), lambda i: (i, 0)),
            pl.BlockSpec((1, d), lambda i: (0, 0)),
        ],
        out_specs=pl.BlockSpec((blk, d), lambda i: (i, 0)),
        out_shape=jax.ShapeDtypeStruct((n, d), jnp.float32),
    )(*parts, hp, dis, b)


def kernel(x, edge_index, W, b):
    n, _ = x.shape
    d = W.shape[1]
    e = edge_index.shape[1]

    ep = -(-e // (NW * CHUNK * WIN)) * (NW * CHUNK * WIN)
    wpw = ep // (NW * WIN)
    np_ = -(-(n + 1) // (NS * WIN)) * (NS * WIN)

    src = edge_index[0]
    dst = edge_index[1]
    src_p = jnp.concatenate(
        [src, jnp.zeros((ep - e,), jnp.int32)]).reshape(NW * wpw, WIN)
    dst_p = jnp.concatenate(
        [dst, jnp.full((ep - e,), n, jnp.int32)]).reshape(NW * wpw, WIN)
    x_p = jnp.pad(x, ((0, np_ - n), (0, 0)))

    hists = _deg_kernel_make(np_, wpw)(dst_p)
    h = _matmul(x_p, W)
    hp, dis = _scale(hists, h)
    parts = _edge_kernel_make(np_, d, wpw)(hp, src_p, dst_p)
    return _finalize(n, parts, hp, dis, b.reshape(1, d))


# speedup vs baseline: 13.2527x; 1.0659x over previous
import functools

import jax
import jax.numpy as jnp
from jax import lax
from jax.experimental import pallas as pl
from jax.experimental.pallas import tpu as pltpu
from jax.experimental.pallas import tpu_sc as plsc

NC = 2
NS = 16
NW = NC * NS
WIN = 128
CHUNK = 40
W0 = 128
W1 = 32
ECH = 32

_MESH = plsc.VectorSubcoreMesh(
    core_axis_name="c", subcore_axis_name="s", num_cores=NC, num_subcores=NS
)


def _deg_kernel_make(np_, wpw):
    stripe = np_ // NS

    @functools.partial(
        pl.kernel,
        out_type=[jax.ShapeDtypeStruct((np_, 16), jnp.float32)
                  for _ in range(NC)],
        mesh=_MESH,
        scratch_types=[
            pltpu.VMEM((wpw, WIN), jnp.int32),
            pltpu.VMEM((WIN, 16), jnp.float32),
            pltpu.VMEM((WIN, 16), jnp.float32),
            pltpu.VMEM_SHARED((np_, 16), jnp.float32),
        ],
    )
    def deg_kernel(dst_hbm, *out_and_scratch):
        outs = out_and_scratch[:NC]
        dstbuf, ones, zeros, hist = out_and_scratch[NC:]
        c = lax.axis_index("c")
        s = lax.axis_index("s")

        @pl.loop(0, WIN)
        def _(j):
            ones[j, :] = jnp.full((16,), 1.0, jnp.float32)
            zeros[j, :] = jnp.zeros((16,), jnp.float32)

        @pl.loop(0, stripe // WIN)
        def _(k):
            pltpu.sync_copy(zeros, hist.at[pl.ds(s * stripe + k * WIN, WIN)])

        base = (c * NS + s) * wpw
        pltpu.sync_copy(dst_hbm.at[pl.ds(base, wpw)], dstbuf)
        plsc.subcore_barrier()

        @pl.loop(0, wpw)
        def _(j):
            pltpu.sync_copy(ones, hist.at[dstbuf.at[j]], add=True)

        plsc.subcore_barrier()
        for ci in range(NC):
            @pl.when(c == ci)
            def _():
                pltpu.sync_copy(hist.at[pl.ds(s * stripe, stripe)],
                                outs[ci].at[pl.ds(s * stripe, stripe)])

    return deg_kernel


def _edge_kernel_make(np_, d):
    stripe = np_ // NS
    assert W0 % ECH == 0 and W1 % ECH == 0 and ECH % 2 == 0

    @functools.partial(
        pl.kernel,
        out_type=[jax.ShapeDtypeStruct((np_, d), jnp.float32)
                  for _ in range(NC)],
        mesh=_MESH,
        scratch_types=[
            pltpu.VMEM((ECH, WIN), jnp.int32),
            pltpu.VMEM((ECH, WIN), jnp.int32),
            pltpu.VMEM((WIN, d), jnp.float32),
            pltpu.VMEM((WIN, d), jnp.float32),
            pltpu.VMEM_SHARED((np_, d), jnp.float32),
            pltpu.SemaphoreType.DMA,
            pltpu.SemaphoreType.DMA,
        ],
    )
    def edge_kernel(hp_hbm, src_hbm, dst_hbm, *out_and_scratch):
        outs = out_and_scratch[:NC]
        srcbuf, dstbuf, rbuf0, rbuf1, acc, sem0, sem1 = out_and_scratch[NC:]
        c = lax.axis_index("c")
        s = lax.axis_index("s")

        pltpu.sync_copy(
            hp_hbm.at[pl.ds(s * stripe, stripe)],
            acc.at[pl.ds(s * stripe, stripe)],
        )
        plsc.subcore_barrier()

        def run_chunk(base):
            pltpu.sync_copy(src_hbm.at[pl.ds(base, ECH)], srcbuf)
            pltpu.sync_copy(dst_hbm.at[pl.ds(base, ECH)], dstbuf)
            pltpu.async_copy(hp_hbm.at[srcbuf.at[0]], rbuf0, sem0)
            pltpu.async_copy(hp_hbm.at[srcbuf.at[1]], rbuf1, sem1)

            @pl.loop(0, ECH, step=2)
            def _(j):
                pltpu.make_async_copy(
                    hp_hbm.at[srcbuf.at[j]], rbuf0, sem0).wait()
                pltpu.sync_copy(rbuf0, acc.at[dstbuf.at[j]], add=True)

                @pl.when(j + 2 < ECH)
                def _():
                    pltpu.async_copy(hp_hbm.at[srcbuf.at[j + 2]], rbuf0, sem0)

                pltpu.make_async_copy(
                    hp_hbm.at[srcbuf.at[j + 1]], rbuf1, sem1).wait()
                pltpu.sync_copy(rbuf1, acc.at[dstbuf.at[j + 1]], add=True)

                @pl.when(j + 3 < ECH)
                def _():
                    pltpu.async_copy(hp_hbm.at[srcbuf.at[j + 3]], rbuf1, sem1)

        @pl.when(c == 0)
        def _():
            for hix in range(W0 // ECH):
                run_chunk(s * W0 + hix * ECH)

        @pl.when(c == 1)
        def _():
            for hix in range(W1 // ECH):
                run_chunk(NS * W0 + s * W1 + hix * ECH)

        plsc.subcore_barrier()
        for ci in range(NC):
            @pl.when(c == ci)
            def _():
                pltpu.sync_copy(acc.at[pl.ds(s * stripe, stripe)],
                                outs[ci].at[pl.ds(s * stripe, stripe)])

    return edge_kernel


def _matmul(x, w):
    m, k = x.shape
    d = w.shape[1]
    blk = 2048

    def body(x_ref, w_ref, o_ref):
        o_ref[...] = jnp.dot(x_ref[...], w_ref[...],
                             preferred_element_type=jnp.float32)

    return pl.pallas_call(
        body,
        grid=(m // blk,),
        in_specs=[
            pl.BlockSpec((blk, k), lambda i: (i, 0)),
            pl.BlockSpec((k, d), lambda i: (0, 0)),
        ],
        out_specs=pl.BlockSpec((blk, d), lambda i: (i, 0)),
        out_shape=jax.ShapeDtypeStruct((m, d), jnp.float32),
    )(x, w)


def _scale(hists, h):
    np_, d = h.shape
    blk = 2048

    def body(*refs):
        hist_refs = refs[:NC]
        h_ref, hp_ref, dis_ref = refs[NC:]
        deg = hist_refs[0][..., 0:1] + 1.0
        for r in hist_refs[1:]:
            deg = deg + r[..., 0:1]
        dis = lax.rsqrt(deg)
        dis_ref[...] = dis
        hp_ref[...] = h_ref[...] * dis

    return pl.pallas_call(
        body,
        grid=(np_ // blk,),
        in_specs=[pl.BlockSpec((blk, 16), lambda i: (i, 0))
                  for _ in range(NC)] + [
            pl.BlockSpec((blk, d), lambda i: (i, 0)),
        ],
        out_specs=[
            pl.BlockSpec((blk, d), lambda i: (i, 0)),
            pl.BlockSpec((blk, 1), lambda i: (i, 0)),
        ],
        out_shape=[
            jax.ShapeDtypeStruct((np_, d), jnp.float32),
            jax.ShapeDtypeStruct((np_, 1), jnp.float32),
        ],
    )(*hists, h)


def _finalize(n, parts, hp, dis, b):
    d = hp.shape[1]
    blk = 2000

    def body(*refs):
        p_refs = refs[:NC]
        hp_ref, dis_ref, b_ref, o_ref = refs[NC:]
        agg = p_refs[0][...]
        for r in p_refs[1:]:
            agg = agg + r[...]
        agg = agg - (NC - 1) * hp_ref[...]
        o_ref[...] = jnp.maximum(dis_ref[...] * agg + b_ref[...], 0.0)

    return pl.pallas_call(
        body,
        grid=(n // blk,),
        in_specs=[pl.BlockSpec((blk, d), lambda i: (i, 0))
                  for _ in range(NC)] + [
            pl.BlockSpec((blk, d), lambda i: (i, 0)),
            pl.BlockSpec((blk, 1), lambda i: (i, 0)),
            pl.BlockSpec((1, d), lambda i: (0, 0)),
        ],
        out_specs=pl.BlockSpec((blk, d), lambda i: (i, 0)),
        out_shape=jax.ShapeDtypeStruct((n, d), jnp.float32),
    )(*parts, hp, dis, b)


def kernel(x, edge_index, W, b):
    n, _ = x.shape
    d = W.shape[1]
    e = edge_index.shape[1]

    ep = NS * (W0 + W1) * WIN
    assert e <= ep
    wpw = ep // (NW * WIN)
    np_ = -(-(n + 1) // (NS * WIN)) * (NS * WIN)

    src = edge_index[0]
    dst = edge_index[1]
    src_p = jnp.concatenate(
        [src, jnp.zeros((ep - e,), jnp.int32)]).reshape(NW * wpw, WIN)
    dst_p = jnp.concatenate(
        [dst, jnp.full((ep - e,), n, jnp.int32)]).reshape(NW * wpw, WIN)
    x_p = jnp.pad(x, ((0, np_ - n), (0, 0)))

    hists = _deg_kernel_make(np_, wpw)(dst_p)
    h = _matmul(x_p, W)
    hp, dis = _scale(hists, h)
    parts = _edge_kernel_make(np_, d)(hp, src_p, dst_p)
    return _finalize(n, parts, hp, dis, b.reshape(1, d))
